# Initial kernel scaffold; baseline (speedup 1.0000x reference)
import jax
import jax.numpy as jnp
from jax.experimental import pallas as pl


def kernel(x, edge_index, W1, b1, W2, b2, W3, b3):
    raise NotImplementedError("write your pallas kernel here")


# speedup vs baseline: 14.0229x; 14.0229x over previous
import functools

import jax
import jax.numpy as jnp
from jax import lax
from jax.experimental import pallas as pl
from jax.experimental.pallas import tpu as pltpu
from jax.experimental.pallas import tpu_sc as plsc

_F32 = jnp.float32
_CH = 128
_NC, _NS = 2, 16
_BLK = 2048


def _ceil_to(v, m):
    return -(-v // m) * m


def _sh(r, k):
    return (r,) if k == 1 else (r, k)


@functools.lru_cache(maxsize=None)
def _sc_scatter(n_pad, e_pad, k, gather):
    rows_pt = n_pad // _NS
    nch = e_pad // (_NC * _NS * _CH)
    mesh = plsc.VectorSubcoreMesh(core_axis_name="c", subcore_axis_name="s")

    def body(u, src, dst, zero, out, sidx, didx, rows, acc, sem):
        c = lax.axis_index("c")
        s = lax.axis_index("s")
        pltpu.sync_copy(zero, acc.at[pl.ds(s * rows_pt, rows_pt)])
        if not gather:
            pltpu.sync_copy(u, rows)
        plsc.subcore_barrier()
        first = (c * _NS + s) * nch

        def step(j, carry):
            base = (first + j) * _CH
            pltpu.sync_copy(dst.at[pl.ds(base, _CH)], didx)
            if gather:
                pltpu.sync_copy(src.at[pl.ds(base, _CH)], sidx)
                pltpu.async_copy(u.at[sidx], rows, sem).wait()
            pltpu.sync_copy(rows, acc.at[didx], add=True)
            return carry

        lax.fori_loop(0, nch, step, 0)
        plsc.subcore_barrier()
        off = s * rows_pt
        pltpu.sync_copy(acc.at[pl.ds(off, rows_pt)],
                        out.at[c, pl.ds(off, rows_pt)])

    return pl.kernel(
        body,
        out_type=jax.ShapeDtypeStruct((_NC,) + _sh(n_pad, k), _F32),
        mesh=mesh,
        compiler_params=pltpu.CompilerParams(use_tc_tiling_on_sc=False),
        scratch_types=[
            pltpu.VMEM((_CH,), jnp.int32),
            pltpu.VMEM((_CH,), jnp.int32),
            pltpu.VMEM(_sh(_CH, k), _F32),
            pltpu.VMEM_SHARED(_sh(n_pad, k), _F32),
            pltpu.SemaphoreType.DMA,
        ],
    )


def _row_spec(k, blk=_BLK):
    return pl.BlockSpec((blk,) + k, lambda i: (i,) + (0,) * len(k))


def _pair_spec(k, blk=_BLK):
    return pl.BlockSpec((_NC, blk) + k, lambda i: (0, i) + (0,) * len(k))


def _full_spec(shape):
    return pl.BlockSpec(shape, lambda i: (0,) * len(shape))


def _tc_a(ind, x):
    def body(ind_ref, x_ref, dis_ref, u0_ref):
        d = lax.rsqrt(ind_ref[0] + ind_ref[1] + 1.0)
        dis_ref[...] = d
        u0_ref[...] = jnp.concatenate(
            [d * x_ref[...], jnp.zeros((x_ref.shape[0], 14), _F32)], axis=1)

    n_pad = x.shape[0]
    return pl.pallas_call(
        body,
        grid=(n_pad // _BLK,),
        in_specs=[_pair_spec((1,)), _row_spec((2,))],
        out_specs=[_row_spec((1,)), _row_spec((16,))],
        out_shape=[jax.ShapeDtypeStruct((n_pad, 1), _F32),
                   jax.ShapeDtypeStruct((n_pad, 16), _F32)],
    )(ind, x)


def _tc_b(s0, u0, dis, W1, b1):
    def body(s0_ref, u0_ref, dis_ref, w_ref, b_ref, p0_ref, ua_ref, ub_ref):
        d = dis_ref[...]
        p0 = d * (s0_ref[0] + s0_ref[1] + u0_ref[...])[:, :2]
        h1 = jnp.maximum(jnp.dot(p0, w_ref[...],
                                 preferred_element_type=_F32) + b_ref[...], 0.0)
        u1 = d * h1
        p0_ref[...] = p0
        ua_ref[...] = u1[:, :16]
        ub_ref[...] = jnp.concatenate(
            [u1[:, 16:], jnp.zeros((u1.shape[0], 2), _F32)], axis=1)

    n_pad = u0.shape[0]
    return pl.pallas_call(
        body,
        grid=(n_pad // _BLK,),
        in_specs=[_pair_spec((16,)), _row_spec((16,)), _row_spec((1,)),
                  _full_spec((2, 30)), _full_spec((1, 30))],
        out_specs=[_row_spec((2,)), _row_spec((16,)), _row_spec((16,))],
        out_shape=[jax.ShapeDtypeStruct((n_pad, 2), _F32),
                   jax.ShapeDtypeStruct((n_pad, 16), _F32),
                   jax.ShapeDtypeStruct((n_pad, 16), _F32)],
    )(s0, u0, dis, W1, b1)


def _tc_c(s1a, s1b, u1a, u1b, dis, p0, x, W2, b2, W3):
    def body(sa_ref, sb_ref, ua_ref, ub_ref, dis_ref, p0_ref, x_ref,
             w2_ref, b2_ref, w3_ref, u3_ref):
        d = dis_ref[...]
        aha = d * (sa_ref[0] + sa_ref[1] + ua_ref[...])
        ahb = d * (sb_ref[0] + sb_ref[1] + ub_ref[...])
        q = jnp.concatenate([aha, ahb[:, :14], p0_ref[...]], axis=1)
        h2 = jnp.maximum(jnp.dot(q, w2_ref[...],
                                 preferred_element_type=_F32) + b2_ref[...],
                         0.0)
        t3 = (jnp.dot(h2, w3_ref[...][:30], preferred_element_type=_F32)
              + jnp.dot(x_ref[...], w3_ref[...][30:],
                        preferred_element_type=_F32))
        u3_ref[...] = d * t3

    n_pad = dis.shape[0]
    return pl.pallas_call(
        body,
        grid=(n_pad // _BLK,),
        in_specs=[_pair_spec((16,)), _pair_spec((16,)), _row_spec((16,)),
                  _row_spec((16,)), _row_spec((1,)), _row_spec((2,)),
                  _row_spec((2,)), _full_spec((32, 30)), _full_spec((1, 30)),
                  _full_spec((32, 1))],
        out_specs=[_row_spec((1,))],
        out_shape=[jax.ShapeDtypeStruct((n_pad, 1), _F32)],
    )(s1a, s1b, u1a, u1b, dis, p0, x, W2, b2, W3)[0]


def _tc_d(s3, u3, dis, b3):
    def body(s3_ref, u3_ref, dis_ref, b3_ref, out_ref):
        out_ref[...] = (dis_ref[...] * (s3_ref[0] + s3_ref[1] + u3_ref[...])
                        + b3_ref[...])

    n_pad = u3.shape[0]
    return pl.pallas_call(
        body,
        grid=(n_pad // _BLK,),
        in_specs=[_pair_spec((1,)), _row_spec((1,)), _row_spec((1,)),
                  _full_spec((1, 1))],
        out_specs=[_row_spec((1,))],
        out_shape=[jax.ShapeDtypeStruct((n_pad, 1), _F32)],
    )(s3, u3, dis, b3)[0]


def kernel(x, edge_index, W1, b1, W2, b2, W3, b3):
    n = x.shape[0]
    e = edge_index.shape[1]
    n_pad = _ceil_to(n + 64, _BLK)
    e_pad = _ceil_to(e, _NC * _NS * _CH)
    spare = n_pad - n

    pad = n + jnp.arange(e_pad - e, dtype=jnp.int32) % spare
    src = jnp.concatenate([edge_index[0], pad])
    dst = jnp.concatenate([edge_index[1], pad])
    x_pad = jnp.zeros((n_pad, 2), _F32).at[:n].set(x)

    rows_pt = n_pad // _NS
    ones_ch = jnp.ones((_CH,), _F32)
    z1 = jnp.zeros((rows_pt,), _F32)
    z16 = jnp.zeros((rows_pt, 16), _F32)

    ind = _sc_scatter(n_pad, e_pad, 1, False)(ones_ch, src, dst, z1)
    dis, u0 = _tc_a(ind.reshape(_NC, n_pad, 1), x_pad)
    s0 = _sc_scatter(n_pad, e_pad, 16, True)(u0, src, dst, z16)
    p0, u1a, u1b = _tc_b(s0, u0, dis, W1, b1.reshape(1, 30))
    s1a = _sc_scatter(n_pad, e_pad, 16, True)(u1a, src, dst, z16)
    s1b = _sc_scatter(n_pad, e_pad, 16, True)(u1b, src, dst, z16)
    u3 = _tc_c(s1a, s1b, u1a, u1b, dis, p0, x_pad, W2,
               b2.reshape(1, 30), W3)
    s3 = _sc_scatter(n_pad, e_pad, 1, True)(u3.reshape(n_pad), src, dst, z1)
    out = _tc_d(s3.reshape(_NC, n_pad, 1), u3, dis, b3.reshape(1, 1))
    return out[:n]


# speedup vs baseline: 28.5070x; 2.0329x over previous
import functools

import jax
import jax.numpy as jnp
from jax import lax
from jax.experimental import pallas as pl
from jax.experimental.pallas import tpu as pltpu
from jax.experimental.pallas import tpu_sc as plsc

_F32 = jnp.float32
_CH = 128
_NB = 4
_NC, _NS = 2, 16
_BLK = 2048


def _ceil_to(v, m):
    return -(-v // m) * m


def _sh(r, k):
    return (r,) if k == 1 else (r, k)


@functools.lru_cache(maxsize=None)
def _sc_scatter(n_pad, e_pad, k, gather, col_split=False):
    rows_pt = n_pad // _NS
    n_tiles = _NS if col_split else _NC * _NS
    nch = e_pad // (n_tiles * _CH)
    nb = nch // _NB
    assert nch % _NB == 0 and nb >= 2
    mesh = plsc.VectorSubcoreMesh(core_axis_name="c", subcore_axis_name="s")

    def body(*refs):
        if col_split:
            ua, ub, src2, dst2, zero, out, sidx, didx, rows, acc, gsem, ssem = refs
        else:
            u, src2, dst2, zero, out, sidx, didx, rows, acc, gsem, ssem = refs
        c = lax.axis_index("c")
        s = lax.axis_index("s")
        pltpu.sync_copy(zero, acc.at[pl.ds(s * rows_pt, rows_pt)])
        if not gather:
            for p in range(2):
                for j in range(_NB):
                    pltpu.sync_copy(u, rows.at[p, j])
        plsc.subcore_barrier()

        def drain_slot(p):
            for j in range(_NB):
                pltpu.make_async_copy(zero.at[pl.ds(0, _CH)],
                                      rows.at[p, j], ssem).wait()

        def run_loop(uu, first):
            def it(b, carry):
                p = jnp.remainder(b, 2)

                @pl.when(b >= 2)
                def _():
                    drain_slot(p)

                row0 = first + b * _NB
                pltpu.sync_copy(dst2.at[pl.ds(row0, _NB)], didx.at[p])
                if gather:
                    pltpu.sync_copy(src2.at[pl.ds(row0, _NB)], sidx.at[p])
                    ds = [pltpu.async_copy(uu.at[sidx.at[p, j]],
                                           rows.at[p, j], gsem)
                          for j in range(_NB)]
                    for d in ds:
                        d.wait()
                for j in range(_NB):
                    pltpu.async_copy(rows.at[p, j], acc.at[didx.at[p, j]],
                                     ssem, add=True)
                return carry

            lax.fori_loop(0, nb, it, 0)

        if col_split:
            first = s * nch

            @pl.when(c == 0)
            def _():
                run_loop(ua, first)

            @pl.when(c == 1)
            def _():
                run_loop(ub, first)
        else:
            run_loop(refs[0], (c * _NS + s) * nch)

        for p in range(2):
            drain_slot(p)
        plsc.subcore_barrier()
        off = s * rows_pt
        pltpu.sync_copy(acc.at[pl.ds(off, rows_pt)],
                        out.at[c, pl.ds(off, rows_pt)])

    return pl.kernel(
        body,
        out_type=jax.ShapeDtypeStruct((_NC,) + _sh(n_pad, k), _F32),
        mesh=mesh,
        compiler_params=pltpu.CompilerParams(use_tc_tiling_on_sc=False),
        scratch_types=[
            pltpu.VMEM((2, _NB, _CH), jnp.int32),
            pltpu.VMEM((2, _NB, _CH), jnp.int32),
            pltpu.VMEM((2, _NB) + _sh(_CH, k), _F32),
            pltpu.VMEM_SHARED(_sh(n_pad, k), _F32),
            pltpu.SemaphoreType.DMA,
            pltpu.SemaphoreType.DMA,
        ],
    )


def _row_spec(k, blk=_BLK):
    return pl.BlockSpec((blk,) + k, lambda i: (i,) + (0,) * len(k))


def _pair_spec(k, blk=_BLK):
    return pl.BlockSpec((_NC, blk) + k, lambda i: (0, i) + (0,) * len(k))


def _full_spec(shape):
    return pl.BlockSpec(shape, lambda i: (0,) * len(shape))


def _tc_a(ind, x):
    def body(ind_ref, x_ref, dis_ref, u0_ref):
        d = lax.rsqrt(ind_ref[0] + ind_ref[1] + 1.0)
        dis_ref[...] = d
        u0_ref[...] = jnp.concatenate(
            [d * x_ref[...], jnp.zeros((x_ref.shape[0], 14), _F32)], axis=1)

    n_pad = x.shape[0]
    return pl.pallas_call(
        body,
        grid=(n_pad // _BLK,),
        in_specs=[_pair_spec((1,)), _row_spec((2,))],
        out_specs=[_row_spec((1,)), _row_spec((16,))],
        out_shape=[jax.ShapeDtypeStruct((n_pad, 1), _F32),
                   jax.ShapeDtypeStruct((n_pad, 16), _F32)],
    )(ind, x)


def _tc_b(s0, u0, dis, W1, b1):
    def body(s0_ref, u0_ref, dis_ref, w_ref, b_ref, p0_ref, ua_ref, ub_ref):
        d = dis_ref[...]
        p0 = d * (s0_ref[0] + s0_ref[1] + u0_ref[...])[:, :2]
        h1 = jnp.maximum(jnp.dot(p0, w_ref[...],
                                 preferred_element_type=_F32) + b_ref[...], 0.0)
        u1 = d * h1
        p0_ref[...] = p0
        ua_ref[...] = u1[:, :16]
        ub_ref[...] = jnp.concatenate(
            [u1[:, 16:], jnp.zeros((u1.shape[0], 2), _F32)], axis=1)

    n_pad = u0.shape[0]
    return pl.pallas_call(
        body,
        grid=(n_pad // _BLK,),
        in_specs=[_pair_spec((16,)), _row_spec((16,)), _row_spec((1,)),
                  _full_spec((2, 30)), _full_spec((1, 30))],
        out_specs=[_row_spec((2,)), _row_spec((16,)), _row_spec((16,))],
        out_shape=[jax.ShapeDtypeStruct((n_pad, 2), _F32),
                   jax.ShapeDtypeStruct((n_pad, 16), _F32),
                   jax.ShapeDtypeStruct((n_pad, 16), _F32)],
    )(s0, u0, dis, W1, b1)


def _tc_c(s1, u1a, u1b, dis, p0, x, W2, b2, W3):
    def body(s1_ref, ua_ref, ub_ref, dis_ref, p0_ref, x_ref,
             w2_ref, b2_ref, w3_ref, u3_ref):
        d = dis_ref[...]
        aha = d * (s1_ref[0] + ua_ref[...])
        ahb = d * (s1_ref[1] + ub_ref[...])
        q = jnp.concatenate([aha, ahb[:, :14], p0_ref[...]], axis=1)
        h2 = jnp.maximum(jnp.dot(q, w2_ref[...],
                                 preferred_element_type=_F32) + b2_ref[...],
                         0.0)
        t3 = (jnp.dot(h2, w3_ref[...][:30], preferred_element_type=_F32)
              + jnp.dot(x_ref[...], w3_ref[...][30:],
                        preferred_element_type=_F32))
        u3_ref[...] = d * t3

    n_pad = dis.shape[0]
    return pl.pallas_call(
        body,
        grid=(n_pad // _BLK,),
        in_specs=[_pair_spec((16,)), _row_spec((16,)), _row_spec((16,)),
                  _row_spec((1,)), _row_spec((2,)), _row_spec((2,)),
                  _full_spec((32, 30)), _full_spec((1, 30)),
                  _full_spec((32, 1))],
        out_specs=[_row_spec((1,))],
        out_shape=[jax.ShapeDtypeStruct((n_pad, 1), _F32)],
    )(s1, u1a, u1b, dis, p0, x, W2, b2, W3)[0]


def _tc_d(s3, u3, dis, b3):
    def body(s3_ref, u3_ref, dis_ref, b3_ref, out_ref):
        out_ref[...] = (dis_ref[...] * (s3_ref[0] + s3_ref[1] + u3_ref[...])
                        + b3_ref[...])

    n_pad = u3.shape[0]
    return pl.pallas_call(
        body,
        grid=(n_pad // _BLK,),
        in_specs=[_pair_spec((1,)), _row_spec((1,)), _row_spec((1,)),
                  _full_spec((1, 1))],
        out_specs=[_row_spec((1,))],
        out_shape=[jax.ShapeDtypeStruct((n_pad, 1), _F32)],
    )(s3, u3, dis, b3)[0]


def kernel(x, edge_index, W1, b1, W2, b2, W3, b3):
    n = x.shape[0]
    e = edge_index.shape[1]
    n_pad = _ceil_to(n + 64, _BLK)
    e_pad = _ceil_to(e, _NC * _NS * _CH * _NB)
    spare = n_pad - n

    pad = n + jnp.arange(e_pad - e, dtype=jnp.int32) % spare
    src = jnp.concatenate([edge_index[0], pad]).reshape(e_pad // _CH, _CH)
    dst = jnp.concatenate([edge_index[1], pad]).reshape(e_pad // _CH, _CH)
    x_pad = jnp.zeros((n_pad, 2), _F32).at[:n].set(x)

    rows_pt = n_pad // _NS
    ones_ch = jnp.ones((_CH,), _F32)
    z1 = jnp.zeros((rows_pt,), _F32)
    z16 = jnp.zeros((rows_pt, 16), _F32)

    ind = _sc_scatter(n_pad, e_pad, 1, False)(ones_ch, src, dst, z1)
    dis, u0 = _tc_a(ind.reshape(_NC, n_pad, 1), x_pad)
    s0 = _sc_scatter(n_pad, e_pad, 16, True)(u0, src, dst, z16)
    p0, u1a, u1b = _tc_b(s0, u0, dis, W1, b1.reshape(1, 30))
    s1 = _sc_scatter(n_pad, e_pad, 16, True, True)(u1a, u1b, src, dst, z16)
    u3 = _tc_c(s1, u1a, u1b, dis, p0, x_pad, W2, b2.reshape(1, 30), W3)
    s3 = _sc_scatter(n_pad, e_pad, 1, True)(u3.reshape(n_pad), src, dst, z1)
    out = _tc_d(s3.reshape(_NC, n_pad, 1), u3, dis, b3.reshape(1, 1))
    return out[:n]
